# baseline (speedup 1.0000x reference)
import functools

import jax
import jax.numpy as jnp
from jax import lax
from jax.experimental import pallas as pl
from jax.experimental.pallas import tpu as pltpu
from jax.experimental.pallas import tpu_sc as plsc

_CH = 2000
_EPS = 1e-6

_pcall = pl.pallas_call


def _leaky(x, slope):
    return jnp.where(x >= 0, x, slope * x)


def _sc_gather(table, idx):
    M = idx.shape[0]
    D = table.shape[1]
    info = plsc.get_sparse_core_info()
    nw = info.num_cores * info.num_subcores
    cs = 5000
    n_chunks = M // cs
    mesh = plsc.VectorSubcoreMesh(core_axis_name="c", subcore_axis_name="s")

    @functools.partial(
        pl.kernel,
        mesh=mesh,
        compiler_params=pltpu.CompilerParams(use_tc_tiling_on_sc=False),
        out_type=jax.ShapeDtypeStruct((M, D), jnp.float32),
        scratch_types=[
            pltpu.VMEM((cs,), jnp.int32),
            pltpu.VMEM((cs, D), jnp.float32),
            pltpu.SemaphoreType.DMA,
        ],
    )
    def k(table_hbm, idx_hbm, out_hbm, idx_v, rows_v, sem):
        wid = lax.axis_index("s") * info.num_cores + lax.axis_index("c")

        def body(j, carry):
            cid = j * nw + wid

            @pl.when(cid < n_chunks)
            def _():
                off = cid * cs
                pltpu.sync_copy(idx_hbm.at[pl.ds(off, cs)], idx_v)
                pltpu.async_copy(table_hbm.at[idx_v], rows_v, sem).wait()
                pltpu.sync_copy(rows_v, out_hbm.at[pl.ds(off, cs)])

            return carry

        lax.fori_loop(0, pl.cdiv(n_chunks, nw), body, 0)

    return k(table, idx)


def _stat_cols(y, c):
    s = jnp.sum(y, 1, keepdims=True)
    ss = jnp.sum(y * y, 1, keepdims=True)
    return jnp.concatenate([s, ss, jnp.zeros((c, 6), jnp.float32)], 1)


def _affine_from(st, g, be, count):
    m = st[:, 0:1] / count
    v = st[:, 1:2] / count - m * m
    a = g / jnp.sqrt(v + _EPS)
    return a, be - a * m


def _bn_affine(st, g, be, count):
    a, c = _affine_from(st, g[:, None], be[:, None], count)
    return a, c


def _acc2(ref, upd, first):
    @pl.when(first)
    def _():
        ref[...] = upd

    @pl.when(jnp.logical_not(first))
    def _():
        ref[...] += upd


def _ka_body(feat_ref, xyz_ref, w1_ref, ws_ref, g1_ref, be1_ref,
             t1_ref, sts_ref, *, N):
    x = feat_ref[...]
    y1 = jnp.dot(w1_ref[...], x, preferred_element_type=jnp.float32)
    a, c = _affine_from(_stat_cols(y1, 8), g1_ref[...], be1_ref[...], N)
    fpc = _leaky(a * y1 + c, 0.2)
    t1_ref[...] = jnp.concatenate([fpc, xyz_ref[...]], axis=0)
    ys = jnp.dot(ws_ref[...], x, preferred_element_type=jnp.float32)
    sts_ref[...] = _stat_cols(ys, 32)


def _fxyz_k(gk, rep, n):
    nx = gk[8:11, :]
    rel = rep - nx
    dist = jnp.sqrt(jnp.sum(rel * rel, axis=0, keepdims=True))
    return jnp.concatenate(
        [dist, rel, rep, nx, jnp.zeros((6, n), jnp.float32)], axis=0)


def _k3_body(g1_ref, xyz_ref, wb1_ref, st_ref):
    first = (pl.program_id(0) == 0) & (pl.program_id(1) == 0)
    gk = jnp.transpose(g1_ref[0])
    fx = _fxyz_k(gk, xyz_ref[0, 0:3, :], _CH)
    yb1 = jnp.dot(wb1_ref[...], fx, preferred_element_type=jnp.float32)
    _acc2(st_ref, _stat_cols(yb1, 8), first)


def _kb_body(g1_ref, xyz_ref, wb1_ref, ab_ref, cb_ref, fct_ref, apw_ref,
             wd_ref, yc_ref, yd_ref, stc_ref, std_ref, m_s, z_s, s_s):
    j = pl.program_id(0)
    k = pl.program_id(1)
    nk = pl.num_programs(1)
    first = (j == 0) & (k == 0)
    gk = jnp.transpose(g1_ref[0])
    fx = _fxyz_k(gk, xyz_ref[0, 0:3, :], _CH)
    yb1 = jnp.dot(wb1_ref[...], fx, preferred_element_type=jnp.float32)
    fxyz1 = _leaky(ab_ref[...] * yb1 + cb_ref[...], 0.2)
    fck = jnp.concatenate([gk[0:8, :], fxyz1], axis=0)
    att = jnp.dot(fct_ref[...], fck, preferred_element_type=jnp.float32)
    ydk = jnp.dot(wd_ref[...], fxyz1, preferred_element_type=jnp.float32)
    yd_ref[0, 0] = ydk
    _acc2(std_ref, _stat_cols(ydk, 8), first)

    @pl.when(k == 0)
    def _():
        m_s[...] = att
        z_s[...] = jnp.ones_like(att)
        s_s[...] = fck

    @pl.when(k > 0)
    def _():
        m_new = jnp.maximum(m_s[...], att)
        r = jnp.exp(m_s[...] - m_new)
        e = jnp.exp(att - m_new)
        z_s[...] = z_s[...] * r + e
        s_s[...] = s_s[...] * r + fck * e
        m_s[...] = m_new

    @pl.when(k == nk - 1)
    def _():
        s1 = s_s[...] / z_s[...]
        yc = jnp.dot(apw_ref[...], s1, preferred_element_type=jnp.float32)
        yc_ref[0] = yc
        _acc2(stc_ref, _stat_cols(yc, 8), j == 0)


def _k5_body(yc_ref, a_ref, c_ref, t2_ref):
    t2_ref[...] = _leaky(a_ref[...] * yc_ref[...] + c_ref[...], 0.2)


def _kc_body(g2_ref, yd_ref, ad_ref, cd_ref, fct_ref, apw_ref,
             ye_ref, ste_ref, m_s, z_s, s_s):
    j = pl.program_id(0)
    k = pl.program_id(1)
    nk = pl.num_programs(1)
    fxyz2 = _leaky(ad_ref[...] * yd_ref[0, 0] + cd_ref[...], 0.2)
    fck = jnp.concatenate([jnp.transpose(g2_ref[0]), fxyz2], axis=0)
    att = jnp.dot(fct_ref[...], fck, preferred_element_type=jnp.float32)

    @pl.when(k == 0)
    def _():
        m_s[...] = att
        z_s[...] = jnp.ones_like(att)
        s_s[...] = fck

    @pl.when(k > 0)
    def _():
        m_new = jnp.maximum(m_s[...], att)
        r = jnp.exp(m_s[...] - m_new)
        e = jnp.exp(att - m_new)
        z_s[...] = z_s[...] * r + e
        s_s[...] = s_s[...] * r + fck * e
        m_s[...] = m_new

    @pl.when(k == nk - 1)
    def _():
        s2 = s_s[...] / z_s[...]
        ye = jnp.dot(apw_ref[...], s2, preferred_element_type=jnp.float32)
        ye_ref[0] = ye
        _acc2(ste_ref, _stat_cols(ye, 16), j == 0)


def _kd_body(ye_ref, ste_ref, gap_ref, bep_ref, w2_ref, g2p_ref, be2p_ref,
             feat_ref, ws_ref, as_ref, cs_ref, out_ref, *, N, K):
    ae, ce = _affine_from(ste_ref[...], gap_ref[...], bep_ref[...], N)
    fpc2 = _leaky(ae * ye_ref[...] + ce, 0.2)
    yf = jnp.dot(w2_ref[...], fpc2, preferred_element_type=jnp.float32)
    af, cf = _affine_from(_stat_cols(yf, 32), g2p_ref[...], be2p_ref[...], N)
    ys = jnp.dot(ws_ref[...], feat_ref[...], preferred_element_type=jnp.float32)
    pre = af * yf + cf + as_ref[...] * ys + cs_ref[...]
    out_ref[...] = _leaky(pre, 0.01)


def kernel(feature, xyz, neigh_idx, p):
    N = feature.shape[2]
    K = neigh_idx.shape[2]
    nj = N // _CH
    grid = (nj, K)

    feat = feature[0, :, :, 0]
    xyz8 = jnp.concatenate(
        [jnp.transpose(xyz[0]), jnp.zeros((5, N), jnp.float32)], 0)
    idxt = jnp.transpose(neigh_idx[0]).astype(jnp.int32).reshape(-1)

    wb1 = jnp.concatenate([p['bb_w1'], jnp.zeros((8, 6), jnp.float32)], 1)

    rows = lambda c: pl.BlockSpec((1, _CH, c), lambda j, k: (k, j, 0))
    chunk3 = lambda c: pl.BlockSpec((1, c, _CH), lambda j, k: (j, 0, 0))
    yd4 = pl.BlockSpec((1, 1, 8, _CH), lambda j, k: (k, j, 0, 0))
    const2 = lambda r, c: pl.BlockSpec((r, c), lambda j, k: (0, 0))
    sds = lambda *s: jax.ShapeDtypeStruct(s, jnp.float32)
    vs = lambda *s: pltpu.VMEM(s, jnp.float32)
    to_chunks = lambda x: jnp.transpose(
        x.reshape(x.shape[0], nj, _CH), (1, 0, 2))
    from_chunks = lambda x: jnp.transpose(
        x, (1, 0, 2)).reshape(x.shape[1], N)

    t1, sts = _pcall(
        functools.partial(_ka_body, N=N),
        out_shape=[sds(16, N), sds(32, 8)],
    )(feat, xyz8, p['w1'], p['ws'], p['g1'][:, None], p['be1'][:, None])
    as_, cs_ = _bn_affine(sts, p['gs'], p['bes'], N)
    xyz8c = to_chunks(xyz8)

    g1 = _sc_gather(jnp.transpose(t1), idxt).reshape(K, N, 16)

    stb1 = _pcall(
        _k3_body, grid=grid,
        in_specs=[rows(16), chunk3(8), const2(8, 16)],
        out_specs=const2(8, 8),
        out_shape=sds(8, 8),
    )(g1, xyz8c, wb1)
    ab1, cb1 = _bn_affine(stb1, p['bb_g1'], p['bb_be1'], N * K)

    yc, yd, stc, std_ = _pcall(
        _kb_body, grid=grid,
        in_specs=[rows(16), chunk3(8), const2(8, 16), const2(8, 1),
                  const2(8, 1), const2(16, 16), const2(8, 16), const2(8, 8)],
        out_specs=[chunk3(8), yd4, const2(8, 8), const2(8, 8)],
        out_shape=[sds(nj, 8, _CH), sds(K, nj, 8, _CH), sds(8, 8), sds(8, 8)],
        scratch_shapes=[vs(16, _CH), vs(16, _CH), vs(16, _CH)],
    )(g1, xyz8c, wb1, ab1, cb1, p['ap1_fc'], p['ap1_w'], p['bb_w2'])
    yc = from_chunks(yc)
    ac, cc = _bn_affine(stc, p['ap1_g'], p['ap1_be'], N)
    ad, cd = _bn_affine(std_, p['bb_g2'], p['bb_be2'], N * K)

    t2 = _pcall(
        _k5_body,
        out_shape=sds(8, N),
    )(yc, ac, cc)

    g2 = _sc_gather(jnp.transpose(t2), idxt).reshape(K, N, 8)

    ye, ste = _pcall(
        _kc_body, grid=grid,
        in_specs=[rows(8), yd4, const2(8, 1), const2(8, 1),
                  const2(16, 16), const2(16, 16)],
        out_specs=[chunk3(16), const2(16, 8)],
        out_shape=[sds(nj, 16, _CH), sds(16, 8)],
        scratch_shapes=[vs(16, _CH), vs(16, _CH), vs(16, _CH)],
    )(g2, yd, ad, cd, p['ap2_fc'], p['ap2_w'])
    ye = from_chunks(ye)

    out_cn = _pcall(
        functools.partial(_kd_body, N=N, K=K),
        out_shape=sds(32, N),
    )(ye, ste, p['ap2_g'][:, None], p['ap2_be'][:, None], p['w2'],
      p['g2'][:, None], p['be2'][:, None], feat, p['ws'], as_, cs_)

    return out_cn[None, :, :, None]


# speedup vs baseline: 1.7452x; 1.7452x over previous
import functools

import jax
import jax.numpy as jnp
from jax import lax
from jax.experimental import pallas as pl
from jax.experimental.pallas import tpu as pltpu
from jax.experimental.pallas import tpu_sc as plsc

_EPS = 1e-6

_pcall = pl.pallas_call


def _leaky(x, slope):
    return jnp.where(x >= 0, x, slope * x)


def _sc_gather(table, idx):
    M = idx.shape[0]
    D = table.shape[1]
    info = plsc.get_sparse_core_info()
    nw = info.num_cores * info.num_subcores
    cs = 5000
    n_chunks = M // cs
    mesh = plsc.VectorSubcoreMesh(core_axis_name="c", subcore_axis_name="s")

    @functools.partial(
        pl.kernel,
        mesh=mesh,
        compiler_params=pltpu.CompilerParams(use_tc_tiling_on_sc=False),
        out_type=jax.ShapeDtypeStruct((M, D), jnp.float32),
        scratch_types=[
            pltpu.VMEM((cs,), jnp.int32),
            pltpu.VMEM((cs, D), jnp.float32),
            pltpu.SemaphoreType.DMA,
        ],
    )
    def k(table_hbm, idx_hbm, out_hbm, idx_v, rows_v, sem):
        wid = lax.axis_index("s") * info.num_cores + lax.axis_index("c")

        def body(j, carry):
            cid = j * nw + wid

            @pl.when(cid < n_chunks)
            def _():
                off = cid * cs
                pltpu.sync_copy(idx_hbm.at[pl.ds(off, cs)], idx_v)
                pltpu.async_copy(table_hbm.at[idx_v], rows_v, sem).wait()
                pltpu.sync_copy(rows_v, out_hbm.at[pl.ds(off, cs)])

            return carry

        lax.fori_loop(0, pl.cdiv(n_chunks, nw), body, 0)

    return k(table, idx)


def _stat_cols(y, c):
    s = jnp.sum(y, 1, keepdims=True)
    ss = jnp.sum(y * y, 1, keepdims=True)
    return jnp.concatenate([s, ss, jnp.zeros((c, 6), jnp.float32)], 1)


def _affine_from(st, g, be, count):
    m = st[:, 0:1] / count
    v = st[:, 1:2] / count - m * m
    a = g / jnp.sqrt(v + _EPS)
    return a, be - a * m


def _bn_affine(st, g, be, count):
    return _affine_from(st, g[:, None], be[:, None], count)


def _acc(ref, upd, first):
    @pl.when(first)
    def _():
        ref[...] = upd

    @pl.when(jnp.logical_not(first))
    def _():
        ref[...] += upd


def _ka_body(feat_ref, xyz_ref, w1_ref, ws_ref, g1_ref, be1_ref,
             t1_ref, sts_ref, *, N):
    x = feat_ref[...]
    y1 = jnp.dot(w1_ref[...], x, preferred_element_type=jnp.float32)
    a, c = _affine_from(_stat_cols(y1, 8), g1_ref[...], be1_ref[...], N)
    fpc = _leaky(a * y1 + c, 0.2)
    t1_ref[...] = jnp.concatenate([fpc, xyz_ref[...]], axis=0)
    ys = jnp.dot(ws_ref[...], x, preferred_element_type=jnp.float32)
    sts_ref[...] = _stat_cols(ys, 32)


def _fxyz_k(gk, rep, n):
    nx = gk[8:11, :]
    rel = rep - nx
    dist = jnp.sqrt(jnp.sum(rel * rel, axis=0, keepdims=True))
    return jnp.concatenate(
        [dist, rel, rep, nx, jnp.zeros((6, n), jnp.float32)], axis=0)


def _k3_body(g1_ref, xyz_ref, wb1_ref, st_ref):
    n = g1_ref.shape[2]
    first = pl.program_id(0) == 0
    fx = _fxyz_k(g1_ref[0], xyz_ref[0:3, :], n)
    yb1 = jnp.dot(wb1_ref[...], fx, preferred_element_type=jnp.float32)
    _acc(st_ref, _stat_cols(yb1, 8), first)


def _k4_body(g1_ref, xyz_ref, wb1_ref, ab_ref, cb_ref, fct_ref, apw_ref,
             wd_ref, yc_ref, yd_ref, stc_ref, std_ref, m_s, z_s, s_s):
    k = pl.program_id(0)
    nk = pl.num_programs(0)
    n = g1_ref.shape[2]
    gk = g1_ref[0]
    fx = _fxyz_k(gk, xyz_ref[0:3, :], n)
    yb1 = jnp.dot(wb1_ref[...], fx, preferred_element_type=jnp.float32)
    fxyz1 = _leaky(ab_ref[...] * yb1 + cb_ref[...], 0.2)
    fck = jnp.concatenate([gk[0:8, :], fxyz1], axis=0)
    att = jnp.dot(fct_ref[...], fck, preferred_element_type=jnp.float32)
    ydk = jnp.dot(wd_ref[...], fxyz1, preferred_element_type=jnp.float32)
    yd_ref[0] = ydk
    _acc(std_ref, _stat_cols(ydk, 8), k == 0)

    @pl.when(k == 0)
    def _():
        m_s[...] = att
        z_s[...] = jnp.ones_like(att)
        s_s[...] = fck

    @pl.when(k > 0)
    def _():
        m_new = jnp.maximum(m_s[...], att)
        r = jnp.exp(m_s[...] - m_new)
        e = jnp.exp(att - m_new)
        z_s[...] = z_s[...] * r + e
        s_s[...] = s_s[...] * r + fck * e
        m_s[...] = m_new

    @pl.when(k == nk - 1)
    def _():
        s1 = s_s[...] / z_s[...]
        yc = jnp.dot(apw_ref[...], s1, preferred_element_type=jnp.float32)
        yc_ref[...] = yc
        stc_ref[...] = _stat_cols(yc, 8)


def _k5_body(yc_ref, a_ref, c_ref, t2_ref):
    t2_ref[...] = _leaky(a_ref[...] * yc_ref[...] + c_ref[...], 0.2)


def _k6_body(g2_ref, yd_ref, ad_ref, cd_ref, fct_ref, apw_ref,
             ye_ref, ste_ref, m_s, z_s, s_s):
    k = pl.program_id(0)
    nk = pl.num_programs(0)
    fxyz2 = _leaky(ad_ref[...] * yd_ref[0] + cd_ref[...], 0.2)
    fck = jnp.concatenate([g2_ref[0], fxyz2], axis=0)
    att = jnp.dot(fct_ref[...], fck, preferred_element_type=jnp.float32)

    @pl.when(k == 0)
    def _():
        m_s[...] = att
        z_s[...] = jnp.ones_like(att)
        s_s[...] = fck

    @pl.when(k > 0)
    def _():
        m_new = jnp.maximum(m_s[...], att)
        r = jnp.exp(m_s[...] - m_new)
        e = jnp.exp(att - m_new)
        z_s[...] = z_s[...] * r + e
        s_s[...] = s_s[...] * r + fck * e
        m_s[...] = m_new

    @pl.when(k == nk - 1)
    def _():
        s2 = s_s[...] / z_s[...]
        ye = jnp.dot(apw_ref[...], s2, preferred_element_type=jnp.float32)
        ye_ref[...] = ye
        ste_ref[...] = _stat_cols(ye, 16)


def _kd_body(ye_ref, ste_ref, gap_ref, bep_ref, w2_ref, g2p_ref, be2p_ref,
             feat_ref, ws_ref, as_ref, cs_ref, out_ref, *, N):
    ae, ce = _affine_from(ste_ref[...], gap_ref[...], bep_ref[...], N)
    fpc2 = _leaky(ae * ye_ref[...] + ce, 0.2)
    yf = jnp.dot(w2_ref[...], fpc2, preferred_element_type=jnp.float32)
    af, cf = _affine_from(_stat_cols(yf, 32), g2p_ref[...], be2p_ref[...], N)
    ys = jnp.dot(ws_ref[...], feat_ref[...], preferred_element_type=jnp.float32)
    pre = af * yf + cf + as_ref[...] * ys + cs_ref[...]
    out_ref[...] = _leaky(pre, 0.01)


def kernel(feature, xyz, neigh_idx, p):
    N = feature.shape[2]
    K = neigh_idx.shape[2]

    feat = feature[0, :, :, 0]
    xyz8 = jnp.concatenate(
        [jnp.transpose(xyz[0]), jnp.zeros((5, N), jnp.float32)], 0)
    idxt = jnp.transpose(neigh_idx[0]).astype(jnp.int32).reshape(-1)

    wb1 = jnp.concatenate([p['bb_w1'], jnp.zeros((8, 6), jnp.float32)], 1)

    plane = lambda c: pl.BlockSpec((1, c, N), lambda k: (k, 0, 0))
    const2 = lambda r, c: pl.BlockSpec((r, c), lambda k: (0, 0))
    sds = lambda *s: jax.ShapeDtypeStruct(s, jnp.float32)
    vs = lambda *s: pltpu.VMEM(s, jnp.float32)

    t1, sts = _pcall(
        functools.partial(_ka_body, N=N),
        out_shape=[sds(16, N), sds(32, 8)],
    )(feat, xyz8, p['w1'], p['ws'], p['g1'][:, None], p['be1'][:, None])
    as_, cs_ = _bn_affine(sts, p['gs'], p['bes'], N)

    g1 = _sc_gather(jnp.transpose(t1), idxt)
    g1t = jnp.transpose(g1.reshape(K, N, 16), (0, 2, 1))

    stb1 = _pcall(
        _k3_body, grid=(K,),
        in_specs=[plane(16), const2(8, N), const2(8, 16)],
        out_specs=const2(8, 8),
        out_shape=sds(8, 8),
    )(g1t, xyz8, wb1)
    ab1, cb1 = _bn_affine(stb1, p['bb_g1'], p['bb_be1'], N * K)

    yc, yd, stc, std_ = _pcall(
        _k4_body, grid=(K,),
        in_specs=[plane(16), const2(8, N), const2(8, 16), const2(8, 1),
                  const2(8, 1), const2(16, 16), const2(8, 16), const2(8, 8)],
        out_specs=[const2(8, N), plane(8), const2(8, 8), const2(8, 8)],
        out_shape=[sds(8, N), sds(K, 8, N), sds(8, 8), sds(8, 8)],
        scratch_shapes=[vs(16, N), vs(16, N), vs(16, N)],
    )(g1t, xyz8, wb1, ab1, cb1, p['ap1_fc'], p['ap1_w'], p['bb_w2'])
    ac, cc = _bn_affine(stc, p['ap1_g'], p['ap1_be'], N)
    ad, cd = _bn_affine(std_, p['bb_g2'], p['bb_be2'], N * K)

    t2 = _pcall(
        _k5_body,
        out_shape=sds(8, N),
    )(yc, ac, cc)

    g2 = _sc_gather(jnp.transpose(t2), idxt)
    g2t = jnp.transpose(g2.reshape(K, N, 8), (0, 2, 1))

    ye, ste = _pcall(
        _k6_body, grid=(K,),
        in_specs=[plane(8), plane(8), const2(8, 1), const2(8, 1),
                  const2(16, 16), const2(16, 16)],
        out_specs=[const2(16, N), const2(16, 8)],
        out_shape=[sds(16, N), sds(16, 8)],
        scratch_shapes=[vs(16, N), vs(16, N), vs(16, N)],
    )(g2t, yd, ad, cd, p['ap2_fc'], p['ap2_w'])

    out_cn = _pcall(
        functools.partial(_kd_body, N=N),
        out_shape=sds(32, N),
    )(ye, ste, p['ap2_g'][:, None], p['ap2_be'][:, None], p['w2'],
      p['g2'][:, None], p['be2'][:, None], feat, p['ws'], as_, cs_)

    return out_cn[None, :, :, None]


# speedup vs baseline: 1.7547x; 1.0054x over previous
import functools

import jax
import jax.numpy as jnp
from jax import lax
from jax.experimental import pallas as pl
from jax.experimental.pallas import tpu as pltpu
from jax.experimental.pallas import tpu_sc as plsc

_EPS = 1e-6

_pcall = pl.pallas_call


def _leaky(x, slope):
    return jnp.where(x >= 0, x, slope * x)


def _sc_gather(table, idx):
    M = idx.shape[0]
    D = table.shape[1]
    info = plsc.get_sparse_core_info()
    nw = info.num_cores * info.num_subcores
    cs = 5000
    n_chunks = M // cs
    mesh = plsc.VectorSubcoreMesh(core_axis_name="c", subcore_axis_name="s")

    @functools.partial(
        pl.kernel,
        mesh=mesh,
        compiler_params=pltpu.CompilerParams(use_tc_tiling_on_sc=False),
        out_type=jax.ShapeDtypeStruct((M, D), jnp.float32),
        scratch_types=[
            pltpu.VMEM((cs,), jnp.int32),
            pltpu.VMEM((cs, D), jnp.float32),
            pltpu.SemaphoreType.DMA,
        ],
    )
    def k(table_hbm, idx_hbm, out_hbm, idx_v, rows_v, sem):
        wid = lax.axis_index("s") * info.num_cores + lax.axis_index("c")

        def body(j, carry):
            cid = j * nw + wid

            @pl.when(cid < n_chunks)
            def _():
                off = cid * cs
                pltpu.sync_copy(idx_hbm.at[pl.ds(off, cs)], idx_v)
                pltpu.async_copy(table_hbm.at[idx_v], rows_v, sem).wait()
                pltpu.sync_copy(rows_v, out_hbm.at[pl.ds(off, cs)])

            return carry

        lax.fori_loop(0, pl.cdiv(n_chunks, nw), body, 0)

    return k(table, idx)


def _stat_cols(y, c):
    s = jnp.sum(y, 1, keepdims=True)
    ss = jnp.sum(y * y, 1, keepdims=True)
    return jnp.concatenate([s, ss, jnp.zeros((c, 6), jnp.float32)], 1)


def _affine_from(st, g, be, count):
    m = st[:, 0:1] / count
    v = st[:, 1:2] / count - m * m
    a = g / jnp.sqrt(v + _EPS)
    return a, be - a * m


def _bn_affine(st, g, be, count):
    return _affine_from(st, g[:, None], be[:, None], count)


def _acc(ref, upd, first):
    @pl.when(first)
    def _():
        ref[...] = upd

    @pl.when(jnp.logical_not(first))
    def _():
        ref[...] += upd


def _ka_body(feat_ref, xyz_ref, w1_ref, ws_ref, g1_ref, be1_ref,
             t1_ref, sts_ref, *, N):
    x = feat_ref[...]
    y1 = jnp.dot(w1_ref[...], x, preferred_element_type=jnp.float32)
    a, c = _affine_from(_stat_cols(y1, 8), g1_ref[...], be1_ref[...], N)
    fpc = _leaky(a * y1 + c, 0.2)
    t1_ref[...] = jnp.concatenate([fpc, xyz_ref[...]], axis=0)
    ys = jnp.dot(ws_ref[...], x, preferred_element_type=jnp.float32)
    sts_ref[...] = _stat_cols(ys, 32)


def _fxyz_k(gk, rep, n):
    nx = gk[8:11, :]
    rel = rep - nx
    dist = jnp.sqrt(jnp.sum(rel * rel, axis=0, keepdims=True))
    return jnp.concatenate(
        [dist, rel, rep, nx, jnp.zeros((6, n), jnp.float32)], axis=0)


def _k3_body(g1_ref, xyz_ref, wb1_ref, st_ref):
    n = g1_ref.shape[2]
    first = pl.program_id(0) == 0
    fx = _fxyz_k(g1_ref[0], xyz_ref[0:3, :], n)
    yb1 = jnp.dot(wb1_ref[...], fx, preferred_element_type=jnp.float32)
    _acc(st_ref, _stat_cols(yb1, 8), first)


def _k4_body(g1_ref, xyz_ref, wb1_ref, ab_ref, cb_ref, fct_ref, apw_ref,
             wd_ref, gc_ref, bec_ref, t2_ref, yd_ref, std_ref,
             m_s, z_s, s_s, *, N):
    k = pl.program_id(0)
    nk = pl.num_programs(0)
    n = g1_ref.shape[2]
    gk = g1_ref[0]
    fx = _fxyz_k(gk, xyz_ref[0:3, :], n)
    yb1 = jnp.dot(wb1_ref[...], fx, preferred_element_type=jnp.float32)
    fxyz1 = _leaky(ab_ref[...] * yb1 + cb_ref[...], 0.2)
    fck = jnp.concatenate([gk[0:8, :], fxyz1], axis=0)
    att = jnp.dot(fct_ref[...], fck, preferred_element_type=jnp.float32)
    ydk = jnp.dot(wd_ref[...], fxyz1, preferred_element_type=jnp.float32)
    yd_ref[0] = ydk
    _acc(std_ref, _stat_cols(ydk, 8), k == 0)

    @pl.when(k == 0)
    def _():
        m_s[...] = att
        z_s[...] = jnp.ones_like(att)
        s_s[...] = fck

    @pl.when(k > 0)
    def _():
        m_new = jnp.maximum(m_s[...], att)
        r = jnp.exp(m_s[...] - m_new)
        e = jnp.exp(att - m_new)
        z_s[...] = z_s[...] * r + e
        s_s[...] = s_s[...] * r + fck * e
        m_s[...] = m_new

    @pl.when(k == nk - 1)
    def _():
        s1 = s_s[...] / z_s[...]
        yc = jnp.dot(apw_ref[...], s1, preferred_element_type=jnp.float32)
        ac, cc = _affine_from(_stat_cols(yc, 8), gc_ref[...], bec_ref[...], N)
        t2_ref[...] = _leaky(ac * yc + cc, 0.2)


def _k6_body(g2_ref, yd_ref, ad_ref, cd_ref, fct_ref, apw_ref,
             gap_ref, bep_ref, w2_ref, g2p_ref, be2p_ref, feat_ref, ws_ref,
             as_ref, cs_ref, out_ref, m_s, z_s, s_s, *, N):
    k = pl.program_id(0)
    nk = pl.num_programs(0)
    fxyz2 = _leaky(ad_ref[...] * yd_ref[0] + cd_ref[...], 0.2)
    fck = jnp.concatenate([g2_ref[0], fxyz2], axis=0)
    att = jnp.dot(fct_ref[...], fck, preferred_element_type=jnp.float32)

    @pl.when(k == 0)
    def _():
        m_s[...] = att
        z_s[...] = jnp.ones_like(att)
        s_s[...] = fck

    @pl.when(k > 0)
    def _():
        m_new = jnp.maximum(m_s[...], att)
        r = jnp.exp(m_s[...] - m_new)
        e = jnp.exp(att - m_new)
        z_s[...] = z_s[...] * r + e
        s_s[...] = s_s[...] * r + fck * e
        m_s[...] = m_new

    @pl.when(k == nk - 1)
    def _():
        s2 = s_s[...] / z_s[...]
        ye = jnp.dot(apw_ref[...], s2, preferred_element_type=jnp.float32)
        ae, ce = _affine_from(_stat_cols(ye, 16), gap_ref[...], bep_ref[...], N)
        fpc2 = _leaky(ae * ye + ce, 0.2)
        yf = jnp.dot(w2_ref[...], fpc2, preferred_element_type=jnp.float32)
        af, cf = _affine_from(
            _stat_cols(yf, 32), g2p_ref[...], be2p_ref[...], N)
        ys = jnp.dot(ws_ref[...], feat_ref[...],
                     preferred_element_type=jnp.float32)
        pre = af * yf + cf + as_ref[...] * ys + cs_ref[...]
        out_ref[...] = _leaky(pre, 0.01)


def kernel(feature, xyz, neigh_idx, p):
    N = feature.shape[2]
    K = neigh_idx.shape[2]

    feat = feature[0, :, :, 0]
    xyz8 = jnp.concatenate(
        [jnp.transpose(xyz[0]), jnp.zeros((5, N), jnp.float32)], 0)
    idxt = jnp.transpose(neigh_idx[0]).astype(jnp.int32).reshape(-1)

    wb1 = jnp.concatenate([p['bb_w1'], jnp.zeros((8, 6), jnp.float32)], 1)

    plane = lambda c: pl.BlockSpec((1, c, N), lambda k: (k, 0, 0))
    const2 = lambda r, c: pl.BlockSpec((r, c), lambda k: (0, 0))
    sds = lambda *s: jax.ShapeDtypeStruct(s, jnp.float32)
    vs = lambda *s: pltpu.VMEM(s, jnp.float32)

    t1, sts = _pcall(
        functools.partial(_ka_body, N=N),
        out_shape=[sds(16, N), sds(32, 8)],
    )(feat, xyz8, p['w1'], p['ws'], p['g1'][:, None], p['be1'][:, None])
    as_, cs_ = _bn_affine(sts, p['gs'], p['bes'], N)

    g1 = _sc_gather(jnp.transpose(t1), idxt)
    g1t = jnp.transpose(g1.reshape(K, N, 16)[:, :, 0:11], (0, 2, 1))

    stb1 = _pcall(
        _k3_body, grid=(K,),
        in_specs=[plane(11), const2(8, N), const2(8, 16)],
        out_specs=const2(8, 8),
        out_shape=sds(8, 8),
    )(g1t, xyz8, wb1)
    ab1, cb1 = _bn_affine(stb1, p['bb_g1'], p['bb_be1'], N * K)

    t2, yd, std_ = _pcall(
        functools.partial(_k4_body, N=N), grid=(K,),
        in_specs=[plane(11), const2(8, N), const2(8, 16), const2(8, 1),
                  const2(8, 1), const2(16, 16), const2(8, 16), const2(8, 8),
                  const2(8, 1), const2(8, 1)],
        out_specs=[const2(8, N), plane(8), const2(8, 8)],
        out_shape=[sds(8, N), sds(K, 8, N), sds(8, 8)],
        scratch_shapes=[vs(16, N), vs(16, N), vs(16, N)],
    )(g1t, xyz8, wb1, ab1, cb1, p['ap1_fc'], p['ap1_w'], p['bb_w2'],
      p['ap1_g'][:, None], p['ap1_be'][:, None])
    ad, cd = _bn_affine(std_, p['bb_g2'], p['bb_be2'], N * K)

    g2 = _sc_gather(jnp.transpose(t2), idxt)
    g2t = jnp.transpose(g2.reshape(K, N, 8), (0, 2, 1))

    out_cn = _pcall(
        functools.partial(_k6_body, N=N), grid=(K,),
        in_specs=[plane(8), plane(8), const2(8, 1), const2(8, 1),
                  const2(16, 16), const2(16, 16), const2(16, 1),
                  const2(16, 1), const2(32, 16), const2(32, 1),
                  const2(32, 1), const2(8, N), const2(32, 8),
                  const2(32, 1), const2(32, 1)],
        out_specs=const2(32, N),
        out_shape=sds(32, N),
        scratch_shapes=[vs(16, N), vs(16, N), vs(16, N)],
    )(g2t, yd, ad, cd, p['ap2_fc'], p['ap2_w'],
      p['ap2_g'][:, None], p['ap2_be'][:, None], p['w2'],
      p['g2'][:, None], p['be2'][:, None], feat, p['ws'], as_, cs_)

    return out_cn[None, :, :, None]
